# Initial kernel scaffold; baseline (speedup 1.0000x reference)
import jax
import jax.numpy as jnp
from jax.experimental import pallas as pl


def kernel(coeffs, edge_index, W0, b0, W1, b1):
    raise NotImplementedError("write your pallas kernel here")


# speedup vs baseline: 69.6086x; 69.6086x over previous
import functools

import numpy as np
import jax
import jax.numpy as jnp
from jax import lax
from jax.experimental import pallas as pl
from jax.experimental.pallas import tpu as pltpu
from jax.experimental.pallas import tpu_sc as plsc

_LOG2 = np.float32(np.log(2.0))
_LANE = 128
_KROWS = 23
_NW = 32
_BN = 1024


def _mlp_body(c_ref, w0_ref, b0_ref, w1_ref, b1_ref, o_ref):
    h = lax.dot_general(c_ref[...], w0_ref[...], (((1,), (1,)), ((), ())),
                        preferred_element_type=jnp.float32)
    h = jnp.maximum(h + b0_ref[...], 0.0)
    x = jnp.sum(h * w1_ref[...], axis=1) + b1_ref[0]
    o_ref[...] = x.reshape(o_ref.shape)


def _combine_body(x_ref, p_ref, o_ref):
    o_ref[...] = x_ref[...] + _LOG2 * (p_ref[0] + p_ref[1])


def _make_sc_agg(n_pad, outer):
    chunk = _KROWS * _LANE
    sl = n_pad // 16
    mesh = plsc.VectorSubcoreMesh(core_axis_name="c", subcore_axis_name="s")

    @functools.partial(
        pl.kernel,
        mesh=mesh,
        out_type=jax.ShapeDtypeStruct((2, n_pad), jnp.float32),
        scratch_types=[
            pltpu.VMEM((_KROWS, _LANE), jnp.int32),
            pltpu.VMEM((_KROWS, _LANE), jnp.int32),
            pltpu.VMEM((_LANE,), jnp.float32),
            pltpu.VMEM_SHARED((n_pad,), jnp.float32),
            pltpu.VMEM_SHARED((n_pad,), jnp.float32),
        ],
    )
    def agg(x_hbm, zeros_hbm, row_hbm, col_hbm, out_hbm,
            row_buf, col_buf, vals, xsp, acc):
        c = lax.axis_index("c")
        s = lax.axis_index("s")
        wid = s * 2 + c
        st = s * sl
        pltpu.sync_copy(x_hbm.at[pl.ds(st, sl)], xsp.at[pl.ds(st, sl)])
        pltpu.sync_copy(zeros_hbm.at[pl.ds(st, sl)], acc.at[pl.ds(st, sl)])
        plsc.subcore_barrier()

        def body(it, carry):
            blk = wid * outer + it
            pltpu.sync_copy(row_hbm.at[blk], row_buf)
            pltpu.sync_copy(col_hbm.at[blk], col_buf)
            for j in range(_KROWS):
                pltpu.sync_copy(xsp.at[col_buf.at[j]], vals)
                pltpu.sync_copy(vals, acc.at[row_buf.at[j]], add=True)
            return carry

        lax.fori_loop(0, outer, body, 0)
        plsc.subcore_barrier()
        pltpu.sync_copy(acc.at[pl.ds(st, sl)], out_hbm.at[c, pl.ds(st, sl)])

    del chunk
    return agg


def kernel(coeffs, edge_index, W0, b0, W1, b1):
    n, d = coeffs.shape
    e = edge_index.shape[1]
    n_pad = (n + 1023) // 1024 * 1024
    rows2d = n_pad // _LANE

    chunk = _KROWS * _LANE
    outer = -(-e // (_NW * chunk))
    e_pad = _NW * outer * chunk

    coeffs_p = jnp.pad(coeffs, ((0, n_pad - n), (0, 0)))
    row = edge_index[0]
    col = edge_index[1]
    pad = e_pad - e
    if pad:
        row = jnp.concatenate([row, jnp.full((pad,), n, dtype=jnp.int32)])
        col = jnp.concatenate([col, jnp.zeros((pad,), dtype=jnp.int32)])
    row3 = row.reshape(_NW * outer, _KROWS, _LANE)
    col3 = col.reshape(_NW * outer, _KROWS, _LANE)
    b0_2d = b0.reshape(1, d)

    x2d = pl.pallas_call(
        _mlp_body,
        grid=(n_pad // _BN,),
        in_specs=[
            pl.BlockSpec((_BN, d), lambda i: (i, 0)),
            pl.BlockSpec((d, d), lambda i: (0, 0)),
            pl.BlockSpec((1, d), lambda i: (0, 0)),
            pl.BlockSpec((1, d), lambda i: (0, 0)),
            pl.BlockSpec(memory_space=pltpu.SMEM),
        ],
        out_specs=pl.BlockSpec((_BN // _LANE, _LANE), lambda i: (i, 0)),
        out_shape=jax.ShapeDtypeStruct((rows2d, _LANE), jnp.float32),
    )(coeffs_p, W0, b0_2d, W1, b1)
    x_flat = x2d.reshape(n_pad)

    zeros = jnp.zeros((n_pad,), jnp.float32)
    partials = _make_sc_agg(n_pad, outer)(x_flat, zeros, row3, col3)
    p3d = partials.reshape(2, rows2d, _LANE)

    out2d = pl.pallas_call(
        _combine_body,
        grid=(rows2d // 8,),
        in_specs=[
            pl.BlockSpec((8, _LANE), lambda i: (i, 0)),
            pl.BlockSpec((2, 8, _LANE), lambda i: (0, i, 0)),
        ],
        out_specs=pl.BlockSpec((8, _LANE), lambda i: (i, 0)),
        out_shape=jax.ShapeDtypeStruct((rows2d, _LANE), jnp.float32),
    )(x2d, p3d)
    return out2d.reshape(n_pad)[:n].reshape(n, 1)
